# Initial kernel scaffold; baseline (speedup 1.0000x reference)
import jax
import jax.numpy as jnp
from jax.experimental import pallas as pl


def kernel(x, edge_index, batch, conv0_W, conv0_b, conv1_W, conv1_b, conv2_W, conv2_b, conv3_W, conv3_b, read0_W, read0_b, read1_W, read1_b):
    raise NotImplementedError("write your pallas kernel here")


# speedup vs baseline: 1.2704x; 1.2704x over previous
import functools

import jax
import jax.numpy as jnp
from jax import lax
from jax.experimental import pallas as pl
from jax.experimental.pallas import tpu as pltpu, tpu_sc as plsc

N = 10000
NP = 10240
E = 320000
EPAD = 327680
C = 128
NWORK = 32
CHUNKS = 80
B = 128
NB = 32
BR = NP // NB
CAPB = EPAD
GRP = EPAD // NWORK // 16


def _mesh():
    return plsc.VectorSubcoreMesh(core_axis_name="c", subcore_axis_name="s")


def _hist_call(srcp, bidp):
    @functools.partial(
        pl.kernel,
        mesh=_mesh(),
        out_type=(jax.ShapeDtypeStruct((2, NP), jnp.float32),
                  jax.ShapeDtypeStruct((2, NWORK * NB), jnp.float32)),
        scratch_types=[
            pltpu.VMEM((CHUNKS, B), jnp.int32),
            pltpu.VMEM((CHUNKS, B), jnp.int32),
            pltpu.VMEM((NP // 16,), jnp.float32),
            pltpu.VMEM((B,), jnp.float32),
            pltpu.SemaphoreType.DMA,
            pltpu.VMEM_SHARED((NP,), jnp.float32),
            pltpu.VMEM_SHARED((NWORK * NB,), jnp.float32),
        ],
    )
    def k(src_hbm, bid_hbm, deg_hbm, hist_hbm, idx_v, bid_v, zbuf_v, ones_v,
          sem, deg_sh, hist_sh):
        cid = lax.axis_index("c")
        sid = lax.axis_index("s")
        wid = cid * 16 + sid
        rpw = NP // 16

        def zb(i, carry):
            zbuf_v[pl.ds(i * 16, 16)] = jnp.zeros((16,), jnp.float32)
            return carry

        lax.fori_loop(0, rpw // 16, zb, 0)
        for i in range(B // 16):
            ones_v[pl.ds(i * 16, 16)] = jnp.ones((16,), jnp.float32)
        pltpu.sync_copy(zbuf_v, deg_sh.at[pl.ds(sid * rpw, rpw)])
        @pl.when(sid < 2)
        def _():
            pltpu.sync_copy(zbuf_v.at[pl.ds(0, NWORK * NB // 2)],
                            hist_sh.at[pl.ds(sid * (NWORK * NB // 2),
                                             NWORK * NB // 2)])
        pltpu.sync_copy(src_hbm.at[wid], idx_v)
        pltpu.sync_copy(bid_hbm.at[wid], bid_v)
        plsc.subcore_barrier()

        def body(j, carry):
            pltpu.async_copy(ones_v, deg_sh.at[idx_v.at[j]], sem, add=True)
            pltpu.async_copy(ones_v, hist_sh.at[bid_v.at[j]], sem, add=True)
            return carry

        lax.fori_loop(0, CHUNKS, body, 0)

        def drain(j, carry):
            pltpu.make_async_copy(ones_v, deg_sh.at[idx_v.at[0]], sem).wait()
            pltpu.make_async_copy(ones_v, hist_sh.at[bid_v.at[0]], sem).wait()
            return carry

        lax.fori_loop(0, CHUNKS, drain, 0)
        plsc.subcore_barrier()
        pltpu.sync_copy(deg_sh.at[pl.ds(sid * rpw, rpw)],
                        deg_hbm.at[cid, pl.ds(sid * rpw, rpw)])
        @pl.when(sid == 0)
        def _():
            pltpu.sync_copy(hist_sh, hist_hbm.at[cid])

    return k(srcp, bidp)


def _lap_call(v, lsrc, lrel, lnorm, tch):
    @functools.partial(
        pl.kernel,
        mesh=_mesh(),
        out_type=jax.ShapeDtypeStruct((NP, C), jnp.float32),
        scratch_types=[
            pltpu.VMEM((BR, C), jnp.float32),
            pltpu.VMEM((2, B, C), jnp.float32),
            pltpu.VMEM((2, B), jnp.int32),
            pltpu.VMEM((2, B), jnp.int32),
            pltpu.VMEM((2, B), jnp.float32),
            pltpu.VMEM((16,), jnp.int32),
            pltpu.SemaphoreType.DMA,
            pltpu.SemaphoreType.DMA,
        ],
    )
    def k(v_hbm, lsrc_hbm, lrel_hbm, lnorm_hbm, tch_hbm, out_hbm,
          acc_v, bufs, sring, rring, nring, tch_v, isem, gsem):
        cid = lax.axis_index("c")
        sid = lax.axis_index("s")
        bkt = cid * 16 + sid

        def za(r, carry):
            for cch in range(C // 16):
                acc_v[r, pl.ds(cch * 16, 16)] = jnp.zeros((16,), jnp.float32)
            return carry

        lax.fori_loop(0, BR, za, 0)
        pltpu.sync_copy(tch_hbm.at[bkt], tch_v)
        nch = tch_v[...][0]

        def fire_idx(t, slot):
            m = t
            pltpu.async_copy(lsrc_hbm.at[bkt, pl.ds(m * B, B)],
                             sring.at[slot], isem)
            pltpu.async_copy(lrel_hbm.at[bkt, pl.ds(m * B, B)],
                             rring.at[slot], isem)
            pltpu.async_copy(lnorm_hbm.at[bkt, pl.ds(m * B, B)],
                             nring.at[slot], isem)

        def wait_idx(slot):
            pltpu.make_async_copy(lsrc_hbm.at[bkt, pl.ds(0, B)],
                                  sring.at[slot], isem).wait()
            pltpu.make_async_copy(lrel_hbm.at[bkt, pl.ds(0, B)],
                                  rring.at[slot], isem).wait()
            pltpu.make_async_copy(lnorm_hbm.at[bkt, pl.ds(0, B)],
                                  nring.at[slot], isem).wait()

        def fire_gather(slot):
            pltpu.async_copy(v_hbm.at[sring.at[slot]], bufs.at[slot], gsem)

        def wait_gather(slot):
            pltpu.make_async_copy(v_hbm.at[sring.at[slot]], bufs.at[slot],
                                  gsem).wait()

        @pl.when(nch > 0)
        def _():
            fire_idx(0, 0)
            wait_idx(0)
            fire_gather(0)

            @pl.when(nch > 1)
            def _():
                fire_idx(1, 1)

        def step(t, slot):
            nslot = 1 - slot
            wait_gather(slot)

            @pl.when(t + 1 < nch)
            def _():
                wait_idx(nslot)
                fire_gather(nslot)

            def rgrp(g, carry2):
                n16 = nring[slot, pl.ds(g * 16, 16)]
                r16 = rring[slot, pl.ds(g * 16, 16)]
                for lane in range(16):
                    rel = r16[lane]
                    nv = jnp.full((16,), n16[lane], jnp.float32)
                    r = g * 16 + lane
                    for cch in range(C // 16):
                        plsc.addupdate(
                            acc_v.at[rel, pl.ds(cch * 16, 16)],
                            nv * bufs[slot, r, pl.ds(cch * 16, 16)])
                return carry2

            lax.fori_loop(0, B // 16, rgrp, 0)

            @pl.when(t + 2 < nch)
            def _():
                fire_idx(t + 2, slot)

        def body(t, carry):
            even = lax.rem(t, 2) == 0

            @pl.when(even)
            def _():
                step(t, 0)

            @pl.when(jnp.logical_not(even))
            def _():
                step(t, 1)

            return carry

        lax.fori_loop(0, nch, body, 0)
        pltpu.sync_copy(acc_v, out_hbm.at[pl.ds(bkt * BR, BR)])

    return k(v, lsrc, lrel, lnorm, tch)


ROWBLK = 1024


def _row_spec():
    return pl.BlockSpec((ROWBLK, C), lambda i: (i, 0))


def _col_spec():
    return pl.BlockSpec((ROWBLK, 1), lambda i: (i, 0))


def _w_spec():
    return pl.BlockSpec((C, C), lambda i: (0, 0))


def _tc_start(h, W):
    def kfn(h_ref, w_ref, acc_ref):
        acc_ref[...] = jnp.dot(h_ref[...], w_ref[...],
                               preferred_element_type=jnp.float32)

    return pl.pallas_call(
        kfn,
        grid=(NP // ROWBLK,),
        in_specs=[_row_spec(), _w_spec()],
        out_specs=[_row_spec()],
        out_shape=[jax.ShapeDtypeStruct((NP, C), jnp.float32)],
    )(h, W)[0]


def _tc_combine(p, v, prev2, diag, acc, W, first):
    def kfn(p_ref, v_ref, prev2_ref, diag_ref, acc_ref, w_ref,
            tx_ref, acc_out_ref):
        lapv = p_ref[...] + diag_ref[...] * v_ref[...]
        if first:
            tx = lapv
        else:
            tx = 2.0 * lapv - prev2_ref[...]
        tx_ref[...] = tx
        acc_out_ref[...] = acc_ref[...] + jnp.dot(
            tx, w_ref[...], preferred_element_type=jnp.float32)

    return pl.pallas_call(
        kfn,
        grid=(NP // ROWBLK,),
        in_specs=[_row_spec(), _row_spec(), _row_spec(), _col_spec(),
                  _row_spec(), _w_spec()],
        out_specs=[_row_spec(), _row_spec()],
        out_shape=[jax.ShapeDtypeStruct((NP, C), jnp.float32)] * 2,
    )(p, v, prev2, diag, acc, W)


def _tc_last(p, v, prev2, diag, acc, W, b):
    def kfn(p_ref, v_ref, prev2_ref, diag_ref, acc_ref, w_ref, b_ref, h_ref):
        lapv = p_ref[...] + diag_ref[...] * v_ref[...]
        tx = 2.0 * lapv - prev2_ref[...]
        out = acc_ref[...] + jnp.dot(tx, w_ref[...],
                                     preferred_element_type=jnp.float32)
        h_ref[...] = jnp.maximum(out + b_ref[...], 0.0)

    return pl.pallas_call(
        kfn,
        grid=(NP // ROWBLK,),
        in_specs=[_row_spec(), _row_spec(), _row_spec(), _col_spec(),
                  _row_spec(), _w_spec(), pl.BlockSpec((1, C), lambda i: (0, 0))],
        out_specs=[_row_spec()],
        out_shape=[jax.ShapeDtypeStruct((NP, C), jnp.float32)],
    )(p, v, prev2, diag, acc, W, b)[0]


def _readout(h, r0W, r0b, r1W, r1b):
    def kfn(h_ref, w0_ref, b0_ref, w1_ref, b1_ref, s_ref):
        t = jnp.maximum(
            jnp.dot(h_ref[...], w0_ref[...],
                    preferred_element_type=jnp.float32) + b0_ref[...], 0.0)
        s = jnp.dot(t, w1_ref[...], preferred_element_type=jnp.float32)
        s_ref[...] = jnp.maximum(s + b1_ref[...], 0.0)

    return pl.pallas_call(
        kfn,
        grid=(NP // ROWBLK,),
        in_specs=[_row_spec(), _w_spec(),
                  pl.BlockSpec((1, C), lambda i: (0, 0)),
                  pl.BlockSpec((C, 1), lambda i: (0, 0)),
                  pl.BlockSpec((1, 1), lambda i: (0, 0))],
        out_specs=[_col_spec()],
        out_shape=[jax.ShapeDtypeStruct((NP, 1), jnp.float32)],
    )(h, r0W, r0b, r1W, r1b)[0]


def _softmax(s2d):
    def kfn(s_ref, o_ref):
        s = s_ref[...]
        row = lax.broadcasted_iota(jnp.int32, s.shape, 0)
        col = lax.broadcasted_iota(jnp.int32, s.shape, 1)
        mask = (row * 128 + col) < N
        m = jnp.max(jnp.where(mask, s, -jnp.inf))
        e = jnp.where(mask, jnp.exp(s - m), 0.0)
        o_ref[...] = e / jnp.sum(e)

    return pl.pallas_call(
        kfn,
        out_shape=jax.ShapeDtypeStruct(s2d.shape, jnp.float32),
    )(s2d)


def kernel(x, edge_index, batch, conv0_W, conv0_b, conv1_W, conv1_b,
           conv2_W, conv2_b, conv3_W, conv3_b, read0_W, read0_b,
           read1_W, read1_b):
    del batch
    src = edge_index[0]
    dst = edge_index[1]
    npad = EPAD - E
    srcp = jnp.concatenate(
        [src, jnp.full((npad,), NP - 1, jnp.int32)]).reshape(NWORK, CHUNKS, B)
    dstp = jnp.concatenate(
        [dst, (jnp.arange(npad, dtype=jnp.int32) % NP)]).reshape(
            NWORK, CHUNKS, B)

    bidp = (jnp.arange(NWORK, dtype=jnp.int32)[:, None, None] * NB
            + dstp // BR)
    degp, histp = _hist_call(srcp, bidp)
    deg = degp[0] + degp[1]
    real = jnp.arange(NP) < N
    dinv = jnp.where(deg > 0, 1.0 / jnp.sqrt(jnp.where(deg > 0, deg, 1.0)),
                     0.0)
    dinv = jnp.where(real, dinv, 0.0).astype(jnp.float32)
    diag = jnp.where(real & (deg <= 0), -1.0, 0.0).astype(
        jnp.float32).reshape(NP, 1)

    srcf = srcp.reshape(-1)
    dstf = dstp.reshape(-1)
    bktf = dstf // BR
    oh = (bktf[:, None] == jnp.arange(NB, dtype=jnp.int32)[None, :]
          ).astype(jnp.int32)
    rank = jnp.take_along_axis(jnp.cumsum(oh, axis=0), bktf[:, None],
                               axis=1)[:, 0] - 1
    cntb = oh.sum(0)
    pos = bktf * CAPB + rank
    normf = -dinv[srcf] * dinv[dstf]
    lsrc = jnp.zeros((NB * CAPB,), jnp.int32).at[pos].set(srcf)
    lsrc = lsrc.reshape(NB, CAPB)
    lrel = jnp.zeros((NB * CAPB,), jnp.int32).at[pos].set(dstf - bktf * BR)
    lrel = lrel.reshape(NB, CAPB)
    lnorm = jnp.zeros((NB * CAPB,), jnp.float32).at[pos].set(normf)
    lnorm = lnorm.reshape(NB, CAPB)
    tch = (cntb + (B - 1)) // B
    tch16 = jnp.zeros((NB, 16), jnp.int32).at[:, 0].set(tch)

    def lap(v):
        return _lap_call(v, lsrc, lrel, lnorm, tch16)

    h = jnp.pad(x, ((0, NP - N), (0, 0)))
    convs = [(conv0_W, conv0_b), (conv1_W, conv1_b), (conv2_W, conv2_b),
             (conv3_W, conv3_b)]
    for (W, bb) in convs:
        acc = _tc_start(h, W[0])
        tx1, acc = _tc_combine(lap(h), h, h, diag, acc, W[1], first=True)
        tx2, acc = _tc_combine(lap(tx1), tx1, h, diag, acc, W[2], first=False)
        tx3, acc = _tc_combine(lap(tx2), tx2, tx1, diag, acc, W[3],
                               first=False)
        h = _tc_last(lap(tx3), tx3, tx2, diag, acc, W[4], bb.reshape(1, C))

    s = _readout(h, read0_W, read0_b.reshape(1, C), read1_W,
                 read1_b.reshape(1, 1))
    sm = _softmax(s.reshape(NP // 128, 128))
    return sm.reshape(NP)[:N]
